# Initial kernel scaffold; baseline (speedup 1.0000x reference)
import jax
import jax.numpy as jnp
from jax.experimental import pallas as pl


def kernel(queries, keys, keys_sym):
    raise NotImplementedError("write your pallas kernel here")


# speedup vs baseline: 1.0506x; 1.0506x over previous
import jax
import jax.numpy as jnp
from jax.experimental import pallas as pl
from jax.experimental.pallas import tpu as pltpu

_Q = 2048
_K = 16384
_D = 64
_KT = 2048
_NT = _K // _KT


def _knn_body(q_ref, k_ref, ks_ref, dis_ref, idx_ref,
              bd_ref, bi_ref, bds_ref, bis_ref):
    j = pl.program_id(0)
    q = q_ref[...]
    qn = jnp.sum(q * q, axis=-1, keepdims=True)
    q2 = -2.0 * q

    def tile_min(k_tile):
        kn = jnp.sum(k_tile * k_tile, axis=-1)
        qk2 = jax.lax.dot_general(
            q2, k_tile, (((1,), (1,)), ((), ())),
            preferred_element_type=jnp.float32)
        d = (qn + kn[None, :]) + qk2
        m = jnp.min(d, axis=1, keepdims=True)
        a = jnp.argmin(d, axis=1).astype(jnp.int32)[:, None] + j * _KT
        return m, a

    m, a = tile_min(k_ref[...])
    ms, as_ = tile_min(ks_ref[...])

    @pl.when(j == 0)
    def _init():
        bd_ref[...], bi_ref[...] = m, a
        bds_ref[...], bis_ref[...] = ms, as_

    @pl.when(j > 0)
    def _update():
        upd = m < bd_ref[...]
        bd_ref[...] = jnp.where(upd, m, bd_ref[...])
        bi_ref[...] = jnp.where(upd, a, bi_ref[...])
        upds = ms < bds_ref[...]
        bds_ref[...] = jnp.where(upds, ms, bds_ref[...])
        bis_ref[...] = jnp.where(upds, as_, bis_ref[...])

    @pl.when(j == _NT - 1)
    def _finish():
        bd, bds = bd_ref[...], bds_ref[...]
        mask = bd < bds
        dis_ref[...] = jnp.where(mask, bd, bds)
        idx_ref[...] = jnp.where(mask, bi_ref[...], bis_ref[...])


def kernel(queries, keys, keys_sym):
    dis, idx = pl.pallas_call(
        _knn_body,
        grid=(_NT,),
        in_specs=[
            pl.BlockSpec((_Q, _D), lambda j: (0, 0)),
            pl.BlockSpec((_KT, _D), lambda j: (j, 0)),
            pl.BlockSpec((_KT, _D), lambda j: (j, 0)),
        ],
        out_specs=[
            pl.BlockSpec((_Q, 1), lambda j: (0, 0)),
            pl.BlockSpec((_Q, 1), lambda j: (0, 0)),
        ],
        out_shape=[
            jax.ShapeDtypeStruct((_Q, 1), jnp.float32),
            jax.ShapeDtypeStruct((_Q, 1), jnp.int32),
        ],
        scratch_shapes=[
            pltpu.VMEM((_Q, 1), jnp.float32),
            pltpu.VMEM((_Q, 1), jnp.int32),
            pltpu.VMEM((_Q, 1), jnp.float32),
            pltpu.VMEM((_Q, 1), jnp.int32),
        ],
        compiler_params=pltpu.CompilerParams(
            dimension_semantics=("arbitrary",)),
    )(queries, keys, keys_sym)
    return dis[:, 0], idx[:, 0]
